# baseline (speedup 1.0000x reference)
import functools

import jax
import jax.numpy as jnp
from jax import lax
from jax.experimental import pallas as pl
from jax.experimental.pallas import tpu as pltpu
from jax.experimental.pallas import tpu_sc as plsc

BATCH = 16384
HIDDEN = 64
NC = 2
NS = 16
NW = NC * NS
BPW = BATCH // NW
CHUNK = 128
NCHUNK = BPW // CHUNK

_mesh = plsc.VectorSubcoreMesh(core_axis_name="c", subcore_axis_name="s")


@functools.partial(
    pl.kernel,
    mesh=_mesh,
    out_type=[
        jax.ShapeDtypeStruct((BATCH, HIDDEN), jnp.float32),
        jax.ShapeDtypeStruct((BATCH, HIDDEN), jnp.float32),
    ],
    scratch_types=[
        pltpu.VMEM((NCHUNK, CHUNK), jnp.int32),
        pltpu.VMEM((NCHUNK, CHUNK), jnp.int32),
        pltpu.VMEM((BPW, HIDDEN), jnp.float32),
        pltpu.VMEM((BPW, HIDDEN), jnp.float32),
        pltpu.SemaphoreType.DMA,
    ],
    compiler_params=pltpu.CompilerParams(use_tc_tiling_on_sc=False),
)
def _sc_gather(uid_hbm, iid_hbm, ut_hbm, it_hbm, uout_hbm, iout_hbm,
               uidx_v, iidx_v, urows_v, irows_v, sem):
    wid = lax.axis_index("s") * NC + lax.axis_index("c")
    base = wid * BPW
    pltpu.sync_copy(uid_hbm.at[wid], uidx_v)
    pltpu.sync_copy(iid_hbm.at[wid], iidx_v)
    copies = []
    for j in range(NCHUNK):
        copies.append(pltpu.async_copy(
            ut_hbm.at[uidx_v.at[j]], urows_v.at[pl.ds(j * CHUNK, CHUNK)], sem))
        copies.append(pltpu.async_copy(
            it_hbm.at[iidx_v.at[j]], irows_v.at[pl.ds(j * CHUNK, CHUNK)], sem))
    for c in copies:
        c.wait()
    pltpu.sync_copy(urows_v, uout_hbm.at[pl.ds(base, BPW)])
    pltpu.sync_copy(irows_v, iout_hbm.at[pl.ds(base, BPW)])


BLK = 2048


def _mlp_body(u_ref, i_ref, w1_ref, b1_ref, w2_ref, b2_ref, w3_ref, b3_ref,
              o_ref):
    u = u_ref[...]
    it = i_ref[...]
    h = jnp.maximum(
        u @ w1_ref[:HIDDEN, :] + it @ w1_ref[HIDDEN:, :] + b1_ref[...], 0.0)
    h = jnp.maximum(h @ w2_ref[...] + b2_ref[...], 0.0)
    o_ref[...] = h @ w3_ref[...] + b3_ref[...]


_mlp = pl.pallas_call(
    _mlp_body,
    grid=(BATCH // BLK,),
    in_specs=[
        pl.BlockSpec((BLK, HIDDEN), lambda n: (n, 0)),
        pl.BlockSpec((BLK, HIDDEN), lambda n: (n, 0)),
        pl.BlockSpec((2 * HIDDEN, HIDDEN), lambda n: (0, 0)),
        pl.BlockSpec((1, HIDDEN), lambda n: (0, 0)),
        pl.BlockSpec((HIDDEN, HIDDEN // 2), lambda n: (0, 0)),
        pl.BlockSpec((1, HIDDEN // 2), lambda n: (0, 0)),
        pl.BlockSpec((HIDDEN // 2, HIDDEN // 4), lambda n: (0, 0)),
        pl.BlockSpec((1, HIDDEN // 4), lambda n: (0, 0)),
    ],
    out_specs=pl.BlockSpec((BLK, HIDDEN // 4), lambda n: (n, 0)),
    out_shape=jax.ShapeDtypeStruct((BATCH, HIDDEN // 4), jnp.float32),
    compiler_params=pltpu.CompilerParams(
        dimension_semantics=("arbitrary",)),
)


def kernel(user_id, item_id, user_table, item_table, W1, b1, W2, b2, W3, b3):
    uid = user_id.astype(jnp.int32).reshape(NW, NCHUNK, CHUNK)
    iid = item_id.astype(jnp.int32).reshape(NW, NCHUNK, CHUNK)
    u_emb, i_emb = _sc_gather(uid, iid, user_table, item_table)
    return _mlp(u_emb, i_emb, W1, b1.reshape(1, -1), W2, b2.reshape(1, -1),
                W3, b3.reshape(1, -1))


# speedup vs baseline: 1.4050x; 1.4050x over previous
import functools

import jax
import jax.numpy as jnp
from jax import lax
from jax.experimental import pallas as pl
from jax.experimental.pallas import tpu as pltpu
from jax.experimental.pallas import tpu_sc as plsc

BATCH = 16384
HIDDEN = 64
SUBL = 8
NC = 2
NS = 16
NW = NC * NS
BPW = BATCH // NW
LANES = 16
NGRP = BPW // LANES

_mesh = plsc.VectorSubcoreMesh(core_axis_name="c", subcore_axis_name="s")


@functools.partial(
    pl.kernel,
    mesh=_mesh,
    out_type=[
        jax.ShapeDtypeStruct((BATCH, HIDDEN), jnp.float32),
        jax.ShapeDtypeStruct((BATCH, HIDDEN), jnp.float32),
    ],
    scratch_types=[
        pltpu.VMEM((BPW,), jnp.int32),
        pltpu.VMEM((BPW,), jnp.int32),
        pltpu.VMEM((BPW,), jnp.int32),
        pltpu.VMEM((BPW,), jnp.int32),
        pltpu.VMEM((LANES * SUBL, HIDDEN), jnp.float32),
        pltpu.VMEM((LANES * SUBL, HIDDEN), jnp.float32),
        pltpu.VMEM((BPW, HIDDEN), jnp.float32),
        pltpu.SemaphoreType.DMA,
        pltpu.SemaphoreType.DMA,
        pltpu.SemaphoreType.DMA,
    ],
    compiler_params=pltpu.CompilerParams(use_tc_tiling_on_sc=True,
                                         needs_layout_passes=False),
)
def _sc_gather(ubase_hbm, ibase_hbm, usub_hbm, isub_hbm, ut_hbm, it_hbm,
               uout_hbm, iout_hbm,
               ubase_v, ibase_v, usub_v, isub_v, gbuf0, gbuf1, selbuf,
               gsem0, gsem1, wsem):
    wid = lax.axis_index("s") * NC + lax.axis_index("c")
    base = wid * BPW
    pltpu.sync_copy(ubase_hbm.at[wid], ubase_v)
    pltpu.sync_copy(ibase_hbm.at[wid], ibase_v)
    pltpu.sync_copy(usub_hbm.at[wid], usub_v)
    pltpu.sync_copy(isub_hbm.at[wid], isub_v)

    lane_iota = lax.iota(jnp.int32, LANES)

    def do_table(tbl, base_ref, sub_ref, out_hbm, selbuf):
        def fire(m, gbuf, gsem):
            vec = base_ref[pl.ds(m * LANES, LANES)]
            for l in range(LANES):
                r0 = pl.multiple_of(vec[l], SUBL)
                pltpu.async_copy(tbl.at[pl.ds(r0, SUBL), :],
                                 gbuf.at[pl.ds(l * SUBL, SUBL), :], gsem)

        def drain(gbuf, gsem):
            pltpu.make_async_copy(tbl.at[pl.ds(0, LANES * SUBL), :],
                                  gbuf, gsem).wait()

        def select(m, gbuf):
            kvec = lane_iota + m * LANES
            subvec = sub_ref[pl.ds(m * LANES, LANES)]
            rvec = lane_iota * SUBL + subvec

            @pl.loop(0, HIDDEN, unroll=4)
            def _(c):
                cvec = jnp.full((LANES,), c, jnp.int32)
                vals = plsc.load_gather(gbuf, [rvec, cvec])
                plsc.store_scatter(selbuf, [kvec, cvec], vals)

        fire(0, gbuf0, gsem0)

        @pl.loop(0, NGRP // 2)
        def _(t):
            m0 = t * 2
            fire(m0 + 1, gbuf1, gsem1)
            drain(gbuf0, gsem0)
            select(m0, gbuf0)

            @pl.when(t < NGRP // 2 - 1)
            def _():
                fire(m0 + 2, gbuf0, gsem0)

            drain(gbuf1, gsem1)
            select(m0 + 1, gbuf1)

        return pltpu.async_copy(selbuf, out_hbm.at[pl.ds(base, BPW), :], wsem)

    cu = do_table(ut_hbm, ubase_v, usub_v, uout_hbm, selbuf)
    cu.wait()
    ci = do_table(it_hbm, ibase_v, isub_v, iout_hbm, selbuf)
    ci.wait()


BLK = 2048


def _mlp_body(u_ref, i_ref, w1_ref, b1_ref, w2_ref, b2_ref, w3_ref, b3_ref,
              o_ref):
    u = u_ref[...]
    it = i_ref[...]
    h = jnp.maximum(
        u @ w1_ref[:HIDDEN, :] + it @ w1_ref[HIDDEN:, :] + b1_ref[...], 0.0)
    h = jnp.maximum(h @ w2_ref[...] + b2_ref[...], 0.0)
    o_ref[...] = h @ w3_ref[...] + b3_ref[...]


_mlp = pl.pallas_call(
    _mlp_body,
    grid=(BATCH // BLK,),
    in_specs=[
        pl.BlockSpec((BLK, HIDDEN), lambda n: (n, 0)),
        pl.BlockSpec((BLK, HIDDEN), lambda n: (n, 0)),
        pl.BlockSpec((2 * HIDDEN, HIDDEN), lambda n: (0, 0)),
        pl.BlockSpec((1, HIDDEN), lambda n: (0, 0)),
        pl.BlockSpec((HIDDEN, HIDDEN // 2), lambda n: (0, 0)),
        pl.BlockSpec((1, HIDDEN // 2), lambda n: (0, 0)),
        pl.BlockSpec((HIDDEN // 2, HIDDEN // 4), lambda n: (0, 0)),
        pl.BlockSpec((1, HIDDEN // 4), lambda n: (0, 0)),
    ],
    out_specs=pl.BlockSpec((BLK, HIDDEN // 4), lambda n: (n, 0)),
    out_shape=jax.ShapeDtypeStruct((BATCH, HIDDEN // 4), jnp.float32),
    compiler_params=pltpu.CompilerParams(
        dimension_semantics=("arbitrary",)),
)


def kernel(user_id, item_id, user_table, item_table, W1, b1, W2, b2, W3, b3):
    uid = user_id.astype(jnp.int32)
    iid = item_id.astype(jnp.int32)
    ubase = (uid & ~jnp.int32(SUBL - 1)).reshape(NW, BPW)
    ibase = (iid & ~jnp.int32(SUBL - 1)).reshape(NW, BPW)
    usub = (uid % SUBL).reshape(NW, BPW)
    isub = (iid % SUBL).reshape(NW, BPW)
    u_emb, i_emb = _sc_gather(ubase, ibase, usub, isub, user_table, item_table)
    return _mlp(u_emb, i_emb, W1, b1.reshape(1, -1), W2, b2.reshape(1, -1),
                W3, b3.reshape(1, -1))
